# baseline (speedup 1.0000x reference)
import functools

import jax
import jax.numpy as jnp
from jax import lax
from jax.experimental import pallas as pl
from jax.experimental.pallas import tpu as pltpu
from jax.experimental.pallas import tpu_sc as plsc

EMB = 32
LANES = 16
CHUNK = 128


def _dot_kernel(b_per_w, num_cores, u_hbm, v_hbm, user_hbm, item_hbm, out_hbm,
                idx_u, idx_v, ue_b, ve_b, ue, ve, out_v, sem):
    wid = lax.axis_index("s") * num_cores + lax.axis_index("c")
    base = wid * b_per_w

    pltpu.sync_copy(u_hbm.at[pl.ds(base, b_per_w)], idx_u)
    pltpu.sync_copy(v_hbm.at[pl.ds(base, b_per_w)], idx_v)

    copies = []
    for c in range(0, b_per_w, CHUNK):
        copies.append(pltpu.async_copy(
            user_hbm.at[idx_u.at[pl.ds(c, CHUNK)]],
            ue_b.at[pl.ds(c, CHUNK), :], sem))
        copies.append(pltpu.async_copy(
            item_hbm.at[idx_v.at[pl.ds(c, CHUNK)]],
            ve_b.at[pl.ds(c, CHUNK), :], sem))
    for cp in copies:
        cp.wait()

    def unpack_body(r, _):
        for src, dst in ((ue_b, ue), (ve_b, ve)):
            row = src[r, :]
            lo, hi = plsc.unpack(row, format=plsc.PackFormat.INTERLEAVED)
            dst[r, pl.ds(0, LANES)] = lo
            dst[r, pl.ds(LANES, LANES)] = hi
        return 0

    lax.fori_loop(0, b_per_w, unpack_body, 0)

    iota = lax.iota(jnp.int32, LANES)

    def block_body(i, _):
        rows = i * LANES + iota
        acc = jnp.zeros((LANES,), jnp.float32)
        for e in range(EMB):
            cols = jnp.full((LANES,), e, jnp.int32)
            gu = plsc.load_gather(ue, [rows, cols])
            gv = plsc.load_gather(ve, [rows, cols])
            acc = acc + gu * gv
        out_v[pl.ds(i * LANES, LANES)] = acc
        return 0

    lax.fori_loop(0, b_per_w // LANES, block_body, 0)

    pltpu.sync_copy(out_v, out_hbm.at[pl.ds(base, b_per_w)])


def kernel(u, v, user_emb, item_emb):
    batch = u.shape[0]
    info = plsc.get_sparse_core_info()
    num_workers = info.num_cores * info.num_subcores
    b_per_w = batch // num_workers

    mesh = plsc.VectorSubcoreMesh(core_axis_name="c", subcore_axis_name="s")
    run = pl.kernel(
        functools.partial(_dot_kernel, b_per_w, info.num_cores),
        mesh=mesh,
        out_type=jax.ShapeDtypeStruct((batch,), jnp.float32),
        scratch_types=[
            pltpu.VMEM((b_per_w,), jnp.int32),
            pltpu.VMEM((b_per_w,), jnp.int32),
            pltpu.VMEM((b_per_w, EMB), jnp.bfloat16),
            pltpu.VMEM((b_per_w, EMB), jnp.bfloat16),
            pltpu.VMEM((b_per_w, EMB), jnp.float32),
            pltpu.VMEM((b_per_w, EMB), jnp.float32),
            pltpu.VMEM((b_per_w,), jnp.float32),
            pltpu.SemaphoreType.DMA,
        ],
        compiler_params=pltpu.CompilerParams(
            needs_layout_passes=False, use_tc_tiling_on_sc=False),
    )
    return run(u, v, user_emb.astype(jnp.bfloat16), item_emb.astype(jnp.bfloat16))


# speedup vs baseline: 1.1766x; 1.1766x over previous
import functools

import jax
import jax.numpy as jnp
from jax import lax
from jax.experimental import pallas as pl
from jax.experimental.pallas import tpu as pltpu
from jax.experimental.pallas import tpu_sc as plsc

EMB = 32
LANES = 16
CHUNK = 128


def _dot_kernel(b_per_w, num_cores, u_hbm, v_hbm, user_hbm, item_hbm, out_hbm,
                idx_u, idx_v, ue, ve, out_v, sem):
    wid = lax.axis_index("s") * num_cores + lax.axis_index("c")
    base = wid * b_per_w

    pltpu.sync_copy(u_hbm.at[pl.ds(base, b_per_w)], idx_u)
    pltpu.sync_copy(v_hbm.at[pl.ds(base, b_per_w)], idx_v)

    copies = []
    for c in range(0, b_per_w, CHUNK):
        copies.append(pltpu.async_copy(
            user_hbm.at[idx_u.at[pl.ds(c, CHUNK)]],
            ue.at[pl.ds(c, CHUNK), :], sem))
        copies.append(pltpu.async_copy(
            item_hbm.at[idx_v.at[pl.ds(c, CHUNK)]],
            ve.at[pl.ds(c, CHUNK), :], sem))
    for cp in copies:
        cp.wait()

    iota = lax.iota(jnp.int32, LANES)

    def block_body(i, _):
        rows = i * LANES + iota
        acc = jnp.zeros((LANES,), jnp.float32)
        for e in range(EMB):
            cols = jnp.full((LANES,), e, jnp.int32)
            gu = plsc.load_gather(ue, [rows, cols])
            gv = plsc.load_gather(ve, [rows, cols])
            acc = acc + gu * gv
        out_v[pl.ds(i * LANES, LANES)] = acc
        return 0

    lax.fori_loop(0, b_per_w // LANES, block_body, 0)

    pltpu.sync_copy(out_v, out_hbm.at[pl.ds(base, b_per_w)])


def kernel(u, v, user_emb, item_emb):
    batch = u.shape[0]
    info = plsc.get_sparse_core_info()
    num_workers = info.num_cores * info.num_subcores
    b_per_w = batch // num_workers

    mesh = plsc.VectorSubcoreMesh(core_axis_name="c", subcore_axis_name="s")
    run = pl.kernel(
        functools.partial(_dot_kernel, b_per_w, info.num_cores),
        mesh=mesh,
        out_type=jax.ShapeDtypeStruct((batch,), jnp.float32),
        scratch_types=[
            pltpu.VMEM((b_per_w,), jnp.int32),
            pltpu.VMEM((b_per_w,), jnp.int32),
            pltpu.VMEM((b_per_w, EMB), jnp.float32),
            pltpu.VMEM((b_per_w, EMB), jnp.float32),
            pltpu.VMEM((b_per_w,), jnp.float32),
            pltpu.SemaphoreType.DMA,
        ],
        compiler_params=pltpu.CompilerParams(
            needs_layout_passes=False, use_tc_tiling_on_sc=False),
    )
    return run(u, v, user_emb, item_emb)
